# baseline (speedup 1.0000x reference)
import functools

import jax
import jax.numpy as jnp
from jax import lax
from jax.experimental import pallas as pl
from jax.experimental.pallas import tpu as pltpu
from jax.experimental.pallas import tpu_sc as plsc

_NC = 2
_NS = 16
_NW = _NC * _NS
_L = 16
_C = 2000


def _lj_body(n_tile, n_chunks, pairs_hbm, tab_hbm, consts_hbm, out_hbm,
             idx2_v, idx_v, rows_v, consts_v, acc_v, sem):
    cid = lax.axis_index("c")
    sid = lax.axis_index("s")
    wid = sid * _NC + cid

    pltpu.sync_copy(consts_hbm, consts_v)
    cv0 = consts_v[pl.ds(0, _L)]
    cv1 = consts_v[pl.ds(8, _L)]

    def cget(i):
        return cv0[i] if i < _L else cv1[i - 8]

    bi = [cget(k) for k in range(9)]
    bx = [cget(9 + k) for k in range(9)]
    cut2 = cget(18)

    lane1 = lax.iota(jnp.int32, _L)
    zero16 = jnp.zeros((_L,), jnp.int32)

    def chunk_body(g, acc):
        base = wid * n_tile + g * _C
        pltpu.sync_copy(pairs_hbm.at[pl.ds(base, _C), :], idx2_v)

        def deint(jj, carry):
            r = jj * 16 + lane1
            v0 = plsc.load_gather(idx2_v, [r, zero16])
            v1 = plsc.load_gather(idx2_v, [r, zero16 + 1])
            idx_v[0, pl.ds(jj * 16, _L)] = v0
            idx_v[1, pl.ds(jj * 16, _L)] = v1
            return carry

        lax.fori_loop(0, _C // _L, deint, 0)
        pltpu.async_copy(tab_hbm.at[idx_v.at[0]], rows_v.at[0], sem).wait()
        pltpu.async_copy(tab_hbm.at[idx_v.at[1]], rows_v.at[1], sem).wait()

        def inner(j, acc):
            r = j * 16 + lane1
            f = [plsc.load_gather(rows_v, [zero16 + e, r, zero16 + k])
                 for e in (0, 1) for k in range(5)]
            x0, y0, z0, s0, e0, x1, y1, z1, s1, e1 = f
            dx = x0 - x1
            dy = y0 - y1
            dz = z0 - z1
            sx = dx * bi[0] + dy * bi[3] + dz * bi[6]
            sy = dx * bi[1] + dy * bi[4] + dz * bi[7]
            sz = dx * bi[2] + dy * bi[5] + dz * bi[8]

            def wrap(s):
                y = s + 0.5
                t = y.astype(jnp.int32).astype(jnp.float32)
                fl = jnp.where(t > y, t - 1.0, t)
                return s - fl

            wx = wrap(sx)
            wy = wrap(sy)
            wz = wrap(sz)
            px = wx * bx[0] + wy * bx[3] + wz * bx[6]
            py = wx * bx[1] + wy * bx[4] + wz * bx[7]
            pz = wx * bx[2] + wy * bx[5] + wz * bx[8]
            r2 = px * px + py * py + pz * pz
            sig = (s0 + s1) * 0.5
            q = (sig * sig) / r2
            t3 = q * q * q
            ene = (4.0 * (e0 * e1)) * (t3 * (t3 - 1.0))
            return acc + jnp.where(r2 <= cut2, ene, 0.0)

        return lax.fori_loop(0, _C // _L, inner, acc)

    acc = lax.fori_loop(0, n_chunks, chunk_body,
                        jnp.zeros((_L,), jnp.float32))
    acc_v[...] = acc
    pltpu.sync_copy(acc_v, out_hbm.at[wid])


@functools.partial(jax.jit, static_argnums=(3,))
def _lj_launch(pairs_flat, tab, consts, n_tile):
    n_chunks = n_tile // _C
    mesh = plsc.VectorSubcoreMesh(core_axis_name="c", subcore_axis_name="s")
    body = functools.partial(_lj_body, n_tile, n_chunks)
    out = pl.kernel(
        body,
        out_type=jax.ShapeDtypeStruct((_NW, _L), jnp.float32),
        mesh=mesh,
        compiler_params=pltpu.CompilerParams(
            needs_layout_passes=False, use_tc_tiling_on_sc=False),
        scratch_types=[
            pltpu.VMEM((_C, 2), jnp.int32),
            pltpu.VMEM((2, _C), jnp.int32),
            pltpu.VMEM((2, _C, 8), jnp.float32),
            pltpu.VMEM((24,), jnp.float32),
            pltpu.VMEM((_L,), jnp.float32),
            pltpu.SemaphoreType.DMA,
        ],
    )(pairs_flat, tab, consts)
    return jnp.sum(out)


def kernel(coords, pairs, box, sigma, epsilon, cutoff):
    n = coords.shape[0]
    p = pairs.shape[0]
    assert p % (_NW * _C) == 0, p
    box = box.astype(jnp.float32)
    box_inv = jnp.linalg.inv(box)
    tab = jnp.concatenate(
        [coords.astype(jnp.float32),
         sigma.astype(jnp.float32)[:, None],
         jnp.sqrt(epsilon.astype(jnp.float32))[:, None],
         jnp.zeros((n, 3), jnp.float32)], axis=1)
    cut2 = (jnp.asarray(cutoff, jnp.float32) ** 2).reshape(1)
    consts = jnp.concatenate(
        [box_inv.reshape(-1), box.reshape(-1), cut2,
         jnp.zeros((5,), jnp.float32)]).astype(jnp.float32)
    return _lj_launch(jnp.asarray(pairs, jnp.int32), tab, consts, p // _NW)


# speedup vs baseline: 1.2106x; 1.2106x over previous
import functools

import jax
import jax.numpy as jnp
from jax import lax
from jax.experimental import pallas as pl
from jax.experimental.pallas import tpu as pltpu
from jax.experimental.pallas import tpu_sc as plsc

_NC = 2
_NS = 16
_NW = _NC * _NS
_L = 16
_C = 2000


def _lj_body(n_tile, n_chunks, pairs_hbm, tab_hbm, consts_hbm, out_hbm,
             idx_v, rows_v, consts_v, acc_v, sem):
    cid = lax.axis_index("c")
    sid = lax.axis_index("s")
    wid = sid * _NC + cid

    pltpu.sync_copy(consts_hbm, consts_v)
    cv0 = consts_v[pl.ds(0, _L)]
    cv1 = consts_v[pl.ds(8, _L)]

    def cget(i):
        return cv0[i] if i < _L else cv1[i - 8]

    bi = [cget(k) for k in range(9)]
    bx = [cget(9 + k) for k in range(9)]
    cut2 = cget(18)

    lane2 = 2 * lax.iota(jnp.int32, _L)
    zero16 = jnp.zeros((_L,), jnp.int32)

    def chunk_body(g, acc):
        base = (wid * n_tile + g * _C) * 2
        pltpu.sync_copy(pairs_hbm.at[pl.ds(base, 2 * _C)], idx_v)
        pltpu.async_copy(tab_hbm.at[idx_v], rows_v, sem).wait()

        def inner(j, acc):
            r0 = j * 32 + lane2
            r1 = r0 + 1
            f = [plsc.load_gather(rows_v, [r, zero16 + k])
                 for r in (r0, r1) for k in range(5)]
            x0, y0, z0, s0, e0, x1, y1, z1, s1, e1 = f
            dx = x0 - x1
            dy = y0 - y1
            dz = z0 - z1
            sx = dx * bi[0] + dy * bi[3] + dz * bi[6]
            sy = dx * bi[1] + dy * bi[4] + dz * bi[7]
            sz = dx * bi[2] + dy * bi[5] + dz * bi[8]

            def wrap(s):
                y = s + 0.5
                t = y.astype(jnp.int32).astype(jnp.float32)
                fl = jnp.where(t > y, t - 1.0, t)
                return s - fl

            wx = wrap(sx)
            wy = wrap(sy)
            wz = wrap(sz)
            px = wx * bx[0] + wy * bx[3] + wz * bx[6]
            py = wx * bx[1] + wy * bx[4] + wz * bx[7]
            pz = wx * bx[2] + wy * bx[5] + wz * bx[8]
            r2 = px * px + py * py + pz * pz
            sig = (s0 + s1) * 0.5
            q = (sig * sig) / r2
            t3 = q * q * q
            ene = (4.0 * (e0 * e1)) * (t3 * (t3 - 1.0))
            return acc + jnp.where(r2 <= cut2, ene, 0.0)

        return lax.fori_loop(0, _C // _L, inner, acc)

    acc = lax.fori_loop(0, n_chunks, chunk_body,
                        jnp.zeros((_L,), jnp.float32))
    acc_v[...] = acc
    pltpu.sync_copy(acc_v, out_hbm.at[wid])


@functools.partial(jax.jit, static_argnums=(3,))
def _lj_launch(pairs_flat, tab, consts, n_tile):
    n_chunks = n_tile // _C
    mesh = plsc.VectorSubcoreMesh(core_axis_name="c", subcore_axis_name="s")
    body = functools.partial(_lj_body, n_tile, n_chunks)
    out = pl.kernel(
        body,
        out_type=jax.ShapeDtypeStruct((_NW, _L), jnp.float32),
        mesh=mesh,
        compiler_params=pltpu.CompilerParams(
            needs_layout_passes=False, use_tc_tiling_on_sc=False),
        scratch_types=[
            pltpu.VMEM((2 * _C,), jnp.int32),
            pltpu.VMEM((2 * _C, 8), jnp.float32),
            pltpu.VMEM((24,), jnp.float32),
            pltpu.VMEM((_L,), jnp.float32),
            pltpu.SemaphoreType.DMA,
        ],
    )(pairs_flat, tab, consts)
    return jnp.sum(out)


def kernel(coords, pairs, box, sigma, epsilon, cutoff):
    n = coords.shape[0]
    p = pairs.shape[0]
    assert p % (_NW * _C) == 0, p
    box = box.astype(jnp.float32)
    box_inv = jnp.linalg.inv(box)
    tab = jnp.concatenate(
        [coords.astype(jnp.float32),
         sigma.astype(jnp.float32)[:, None],
         jnp.sqrt(epsilon.astype(jnp.float32))[:, None],
         jnp.zeros((n, 3), jnp.float32)], axis=1)
    cut2 = (jnp.asarray(cutoff, jnp.float32) ** 2).reshape(1)
    consts = jnp.concatenate(
        [box_inv.reshape(-1), box.reshape(-1), cut2,
         jnp.zeros((5,), jnp.float32)]).astype(jnp.float32)
    rt_zero = jnp.minimum(jnp.asarray(cutoff, jnp.float32) ** 2, 0.0)
    pairs_flat = pairs.astype(jnp.int32).reshape(-1) + rt_zero.astype(jnp.int32)
    return _lj_launch(pairs_flat, tab, consts, p // _NW)


# speedup vs baseline: 11.3837x; 9.4037x over previous
import functools

import jax
import jax.numpy as jnp
from jax import lax
from jax.experimental import pallas as pl
from jax.experimental.pallas import tpu as pltpu
from jax.experimental.pallas import tpu_sc as plsc

_NC = 2
_NS = 16
_NW = _NC * _NS
_L = 16
_B = 128
_CB = 16
_CP = _B * _CB
_CW = 2 * _CP


def _lj_body(n_chunks, pairs_hbm, tab_hbm, consts_hbm, out_hbm,
             idx_v, rows_v, consts_v, acc_v, sem):
    cid = lax.axis_index("c")
    sid = lax.axis_index("s")
    wid = sid * _NC + cid

    pltpu.sync_copy(consts_hbm, consts_v)
    cv0 = consts_v[pl.ds(0, _L)]
    cv1 = consts_v[pl.ds(8, _L)]

    def cget(i):
        return cv0[i] if i < _L else cv1[i - 8]

    bi = [cget(k) for k in range(9)]
    bx = [cget(9 + k) for k in range(9)]
    cut2 = cget(18)

    lane1 = lax.iota(jnp.int32, _L)
    zero16 = jnp.zeros((_L,), jnp.int32)

    my_chunks = (n_chunks - 1 - wid) // _NW + 1

    def chunk_body(i, acc):
        g = i * _NW + wid
        pltpu.sync_copy(pairs_hbm.at[pl.ds(g * _CW, _CW)], idx_v)
        pltpu.async_copy(tab_hbm.at[idx_v], rows_v, sem).wait()

        def inner(j, acc):
            r0 = (j >> 3) * 256 + (j & 7) * 16 + lane1
            r1 = r0 + 128
            f = [plsc.load_gather(rows_v, [r, zero16 + k])
                 for r in (r0, r1) for k in range(5)]
            x0, y0, z0, s0, e0, x1, y1, z1, s1, e1 = f
            dx = x0 - x1
            dy = y0 - y1
            dz = z0 - z1
            sx = dx * bi[0] + dy * bi[3] + dz * bi[6]
            sy = dx * bi[1] + dy * bi[4] + dz * bi[7]
            sz = dx * bi[2] + dy * bi[5] + dz * bi[8]

            def wrap(s):
                y = s + 0.5
                t = y.astype(jnp.int32).astype(jnp.float32)
                fl = jnp.where(t > y, t - 1.0, t)
                return s - fl

            wx = wrap(sx)
            wy = wrap(sy)
            wz = wrap(sz)
            px = wx * bx[0] + wy * bx[3] + wz * bx[6]
            py = wx * bx[1] + wy * bx[4] + wz * bx[7]
            pz = wx * bx[2] + wy * bx[5] + wz * bx[8]
            r2 = px * px + py * py + pz * pz
            sig = (s0 + s1) * 0.5
            q = (sig * sig) / r2
            t3 = q * q * q
            ene = (4.0 * (e0 * e1)) * (t3 * (t3 - 1.0))
            return acc + jnp.where(r2 <= cut2, ene, 0.0)

        return lax.fori_loop(0, _CP // _L, inner, acc)

    acc = lax.fori_loop(0, my_chunks, chunk_body,
                        jnp.zeros((_L,), jnp.float32))
    acc_v[...] = acc
    pltpu.sync_copy(acc_v, out_hbm.at[wid])


@functools.partial(jax.jit, static_argnums=(3,))
def _lj_launch(pairs_lin, tab, consts, n_chunks):
    mesh = plsc.VectorSubcoreMesh(core_axis_name="c", subcore_axis_name="s")
    body = functools.partial(_lj_body, n_chunks)
    out = pl.kernel(
        body,
        out_type=jax.ShapeDtypeStruct((_NW, _L), jnp.float32),
        mesh=mesh,
        compiler_params=pltpu.CompilerParams(
            needs_layout_passes=False, use_tc_tiling_on_sc=False),
        scratch_types=[
            pltpu.VMEM((_CW,), jnp.int32),
            pltpu.VMEM((_CW, 8), jnp.float32),
            pltpu.VMEM((24,), jnp.float32),
            pltpu.VMEM((_L,), jnp.float32),
            pltpu.SemaphoreType.DMA,
        ],
    )(pairs_lin, tab, consts)
    return jnp.sum(out)


def kernel(coords, pairs, box, sigma, epsilon, cutoff):
    n = coords.shape[0]
    p = pairs.shape[0]
    nb = p // _B
    assert p % _B == 0 and nb % _CB == 0, p
    box = box.astype(jnp.float32)
    box_inv = jnp.linalg.inv(box)
    tab = jnp.concatenate(
        [coords.astype(jnp.float32),
         sigma.astype(jnp.float32)[:, None],
         jnp.sqrt(epsilon.astype(jnp.float32))[:, None],
         jnp.zeros((n, 3), jnp.float32)], axis=1)
    cut2 = (jnp.asarray(cutoff, jnp.float32) ** 2).reshape(1)
    consts = jnp.concatenate(
        [box_inv.reshape(-1), box.reshape(-1), cut2,
         jnp.zeros((5,), jnp.float32)]).astype(jnp.float32)
    pairs_lin = (pairs.astype(jnp.int32).T
                 .reshape(2, nb, _B).transpose(1, 0, 2).reshape(-1))
    return _lj_launch(pairs_lin, tab, consts, nb // _CB)
